# Initial kernel scaffold; baseline (speedup 1.0000x reference)
import jax
import jax.numpy as jnp
from jax.experimental import pallas as pl


def kernel(nodes, item_history, itemrating_history, user_table, item_table, rating_table, ln1_w, ln1_b, ln2_w, ln2_b, ln3_w, ln3_b, att1_w, att1_b, att2_w, att2_b, att3_w, att3_b):
    raise NotImplementedError("write your pallas kernel here")


# speedup vs baseline: 1.7179x; 1.7179x over previous
import jax
import jax.numpy as jnp
from jax.experimental import pallas as pl
from jax.experimental.pallas import tpu as pltpu
from jax.experimental.pallas import tpu_sc as plsc


_NB = 128


def _sc_gather(table, idx, window):
    n = idx.shape[0]
    d = table.shape[1]
    mesh = plsc.VectorSubcoreMesh(core_axis_name="core", subcore_axis_name="subcore")

    @pl.kernel(out_type=jax.ShapeDtypeStruct((n, d), table.dtype), mesh=mesh)
    def kern(tab_hbm, i_hbm, o_hbm):
        def body(i_vmem, o_vmem):
            pltpu.sync_copy(tab_hbm.at[i_vmem.at[0]], o_vmem)

        pltpu.emit_pipeline(
            body,
            grid=(n // window,),
            in_specs=[pl.BlockSpec((1, window), lambda i: (0, i))],
            out_specs=[pl.BlockSpec((window, d), lambda i: (i, 0))],
            core_axis_name=("core", "subcore"),
            dimension_semantics=(pltpu.PARALLEL,),
        )(i_hbm, o_hbm)

    return kern(table, idx.reshape(1, n))


def _tc_body(gp_ref, ipar_ref, rat_ref, up_ref, npar_ref, rtab_ref,
             ln1w_ref, ln1b_ref, ln2w_ref, ln2b_ref, ln3w_ref, ln3b_ref,
             att1w_ref, att1b_ref, att2w_ref, att2b_ref, att3w_ref,
             out_ref):
    f32 = jnp.float32
    hi = jax.lax.Precision.HIGHEST

    def dot_t(x, w):
        return jax.lax.dot_general(x, w, (((1,), (1,)), ((), ())),
                                   precision=hi, preferred_element_type=f32)

    L, nB, D2 = gp_ref.shape
    D = D2 // 2
    gp = gp_ref[...]
    g3 = jnp.where((ipar_ref[...] & 1) == 0, gp[:, :, :D], gp[:, :, D:])
    g2 = g3.reshape(L * nB, D)
    ln1w = ln1w_ref[...]
    item_part = dot_t(g2, ln1w[:, :D])
    pre_rat = dot_t(rtab_ref[...], ln1w[:, D:]) + ln1b_ref[...]
    rat = rat_ref[...]
    ratc = jnp.zeros((L, nB, D), f32)
    for k in range(5):
        ratc = ratc + jnp.where(rat == k, pre_rat[k:k + 1][None], 0.0)
    xr3 = jnp.maximum(item_part.reshape(L, nB, D) + ratc, 0.0)
    xr2 = xr3.reshape(L * nB, D)

    up = up_ref[...]
    u = jnp.where((npar_ref[...] & 1) == 0, up[:, :D], up[:, D:])
    att1w = att1w_ref[...]
    u_att = dot_t(u, att1w[:, D:]) + att1b_ref[...]
    a1 = dot_t(xr2, att1w[:, :D]).reshape(L, nB, D) + u_att[None]
    a1 = jnp.maximum(a1, 0.0)
    a2 = jnp.maximum(dot_t(a1.reshape(L * nB, D), att2w_ref[...])
                     + att2b_ref[...], 0.0)

    s = jnp.sum(a2.reshape(L, nB, D) * att3w_ref[...][None], axis=2,
                keepdims=True)
    m = jnp.max(s, axis=0, keepdims=True)
    e = jnp.exp(s - m)
    denom = jnp.sum(e, axis=0, keepdims=True)
    hI = jnp.sum(xr3 * (e / denom), axis=0)

    h2 = jnp.maximum(dot_t(hI, ln2w_ref[...]) + ln2b_ref[...], 0.0)
    ln3w = ln3w_ref[...]
    out = dot_t(u, ln3w[:, :D]) + dot_t(h2, ln3w[:, D:]) + ln3b_ref[...]
    out_ref[...] = jnp.maximum(out, 0.0)


def _tc_compute(gp3, ipar, rat, up, npar, rating_table,
                ln1_w, ln1_b, ln2_w, ln2_b, ln3_w, ln3_b,
                att1_w, att1_b, att2_w, att2_b, att3_w):
    L, B, D2 = gp3.shape
    grid = (B // _NB,)
    full = lambda shape: pl.BlockSpec(shape, lambda i: tuple(0 for _ in shape))
    return pl.pallas_call(
        _tc_body,
        grid=grid,
        in_specs=[
            pl.BlockSpec((L, _NB, D2), lambda i: (0, i, 0)),
            pl.BlockSpec((L, _NB, 1), lambda i: (0, i, 0)),
            pl.BlockSpec((L, _NB, 1), lambda i: (0, i, 0)),
            pl.BlockSpec((_NB, D2), lambda i: (i, 0)),
            pl.BlockSpec((_NB, 1), lambda i: (i, 0)),
            full(rating_table.shape),
            full(ln1_w.shape), full(ln1_b.shape),
            full(ln2_w.shape), full(ln2_b.shape),
            full(ln3_w.shape), full(ln3_b.shape),
            full(att1_w.shape), full(att1_b.shape),
            full(att2_w.shape), full(att2_b.shape),
            full(att3_w.shape),
        ],
        out_specs=pl.BlockSpec((_NB, D2 // 2), lambda i: (i, 0)),
        out_shape=jax.ShapeDtypeStruct((B, D2 // 2), jnp.float32),
        compiler_params=pltpu.CompilerParams(
            dimension_semantics=("parallel",)),
    )(gp3, ipar, rat, up, npar, rating_table, ln1_w, ln1_b, ln2_w, ln2_b,
      ln3_w, ln3_b, att1_w, att1_b, att2_w, att2_b, att3_w)


def kernel(nodes, item_history, itemrating_history, user_table, item_table,
           rating_table, ln1_w, ln1_b, ln2_w, ln2_b, ln3_w, ln3_b,
           att1_w, att1_b, att2_w, att2_b, att3_w, att3_b):
    B, L = item_history.shape
    D = user_table.shape[1]

    idx_items = item_history.T.reshape(-1).astype(jnp.int32)
    idx_nodes = nodes.astype(jnp.int32)
    item_pairs = item_table.reshape(-1, 2 * D)
    user_pairs = user_table.reshape(-1, 2 * D)
    gp = _sc_gather(item_pairs, idx_items >> 1, 256)
    up = _sc_gather(user_pairs, idx_nodes >> 1, 128)

    gp3 = gp.reshape(L, B, 2 * D)
    ipar = idx_items.reshape(L, B, 1)
    rat = itemrating_history.T.astype(jnp.int32).reshape(L, B, 1)
    npar = idx_nodes.reshape(B, 1)

    r1 = lambda b: b.reshape(1, -1)
    return _tc_compute(gp3, ipar, rat, up, npar, rating_table,
                       ln1_w, r1(ln1_b), ln2_w, r1(ln2_b), ln3_w, r1(ln3_b),
                       att1_w, r1(att1_b), att2_w, r1(att2_b), att3_w)


# speedup vs baseline: 2.2480x; 1.3086x over previous
import jax
import jax.numpy as jnp
from jax.experimental import pallas as pl
from jax.experimental.pallas import tpu as pltpu
from jax.experimental.pallas import tpu_sc as plsc


_NB = 128


def _sc_gather(table, idx, window):
    n = idx.shape[0]
    d = table.shape[1]
    mesh = plsc.VectorSubcoreMesh(core_axis_name="core", subcore_axis_name="subcore")

    @pl.kernel(out_type=jax.ShapeDtypeStruct((n, d), table.dtype), mesh=mesh)
    def kern(tab_hbm, i_hbm, o_hbm):
        def body(i_vmem, o_vmem):
            pltpu.sync_copy(tab_hbm.at[i_vmem.at[0]], o_vmem)

        pltpu.emit_pipeline(
            body,
            grid=(n // window,),
            in_specs=[pl.BlockSpec((1, window), lambda i: (0, i))],
            out_specs=[pl.BlockSpec((window, d), lambda i: (i, 0))],
            core_axis_name=("core", "subcore"),
            dimension_semantics=(pltpu.PARALLEL,),
        )(i_hbm, o_hbm)

    return kern(table, idx.reshape(1, n))


def _tc_body(gp_ref, ipar_ref, rat_ref, up_ref, npar_ref, rtab_ref,
             ln1w_ref, ln1b_ref, ln2w_ref, ln2b_ref, ln3w_ref, ln3b_ref,
             att1w_ref, att1b_ref, att2w_ref, att2b_ref, att3w_ref,
             out_ref):
    f32 = jnp.float32
    hi = jax.lax.Precision.DEFAULT

    def dot_t(x, w):
        return jax.lax.dot_general(x, w, (((1,), (1,)), ((), ())),
                                   precision=hi, preferred_element_type=f32)

    L, nB, D2 = gp_ref.shape
    D = D2 // 2
    gp = gp_ref[...]
    g3 = jnp.where((ipar_ref[...] & 1) == 0, gp[:, :, :D], gp[:, :, D:])
    g2 = g3.reshape(L * nB, D)
    ln1w = ln1w_ref[...]
    item_part = dot_t(g2, ln1w[:, :D])
    pre_rat = dot_t(rtab_ref[...], ln1w[:, D:]) + ln1b_ref[...]
    rat = rat_ref[...]
    ratc = jnp.zeros((L, nB, D), f32)
    for k in range(5):
        ratc = ratc + jnp.where(rat == k, pre_rat[k:k + 1][None], 0.0)
    xr3 = jnp.maximum(item_part.reshape(L, nB, D) + ratc, 0.0)
    xr2 = xr3.reshape(L * nB, D)

    up = up_ref[...]
    u = jnp.where((npar_ref[...] & 1) == 0, up[:, :D], up[:, D:])
    att1w = att1w_ref[...]
    u_att = dot_t(u, att1w[:, D:]) + att1b_ref[...]
    a1 = dot_t(xr2, att1w[:, :D]).reshape(L, nB, D) + u_att[None]
    a1 = jnp.maximum(a1, 0.0)
    a2 = jnp.maximum(dot_t(a1.reshape(L * nB, D), att2w_ref[...])
                     + att2b_ref[...], 0.0)

    s = jnp.sum(a2.reshape(L, nB, D) * att3w_ref[...][None], axis=2,
                keepdims=True)
    m = jnp.max(s, axis=0, keepdims=True)
    e = jnp.exp(s - m)
    denom = jnp.sum(e, axis=0, keepdims=True)
    hI = jnp.sum(xr3 * (e / denom), axis=0)

    h2 = jnp.maximum(dot_t(hI, ln2w_ref[...]) + ln2b_ref[...], 0.0)
    ln3w = ln3w_ref[...]
    out = dot_t(u, ln3w[:, :D]) + dot_t(h2, ln3w[:, D:]) + ln3b_ref[...]
    out_ref[...] = jnp.maximum(out, 0.0)


def _tc_compute(gp3, ipar, rat, up, npar, rating_table,
                ln1_w, ln1_b, ln2_w, ln2_b, ln3_w, ln3_b,
                att1_w, att1_b, att2_w, att2_b, att3_w):
    L, B, D2 = gp3.shape
    grid = (B // _NB,)
    full = lambda shape: pl.BlockSpec(shape, lambda i: tuple(0 for _ in shape))
    return pl.pallas_call(
        _tc_body,
        grid=grid,
        in_specs=[
            pl.BlockSpec((L, _NB, D2), lambda i: (0, i, 0)),
            pl.BlockSpec((L, _NB, 1), lambda i: (0, i, 0)),
            pl.BlockSpec((L, _NB, 1), lambda i: (0, i, 0)),
            pl.BlockSpec((_NB, D2), lambda i: (i, 0)),
            pl.BlockSpec((_NB, 1), lambda i: (i, 0)),
            full(rating_table.shape),
            full(ln1_w.shape), full(ln1_b.shape),
            full(ln2_w.shape), full(ln2_b.shape),
            full(ln3_w.shape), full(ln3_b.shape),
            full(att1_w.shape), full(att1_b.shape),
            full(att2_w.shape), full(att2_b.shape),
            full(att3_w.shape),
        ],
        out_specs=pl.BlockSpec((_NB, D2 // 2), lambda i: (i, 0)),
        out_shape=jax.ShapeDtypeStruct((B, D2 // 2), jnp.float32),
        compiler_params=pltpu.CompilerParams(
            dimension_semantics=("parallel",)),
    )(gp3, ipar, rat, up, npar, rating_table, ln1_w, ln1_b, ln2_w, ln2_b,
      ln3_w, ln3_b, att1_w, att1_b, att2_w, att2_b, att3_w)


def kernel(nodes, item_history, itemrating_history, user_table, item_table,
           rating_table, ln1_w, ln1_b, ln2_w, ln2_b, ln3_w, ln3_b,
           att1_w, att1_b, att2_w, att2_b, att3_w, att3_b):
    B, L = item_history.shape
    D = user_table.shape[1]

    idx_items = item_history.T.reshape(-1).astype(jnp.int32)
    idx_nodes = nodes.astype(jnp.int32)
    item_pairs = item_table.reshape(-1, 2 * D)
    user_pairs = user_table.reshape(-1, 2 * D)
    gp = _sc_gather(item_pairs, idx_items >> 1, 256)
    up = _sc_gather(user_pairs, idx_nodes >> 1, 128)

    gp3 = gp.reshape(L, B, 2 * D)
    ipar = idx_items.reshape(L, B, 1)
    rat = itemrating_history.T.astype(jnp.int32).reshape(L, B, 1)
    npar = idx_nodes.reshape(B, 1)

    r1 = lambda b: b.reshape(1, -1)
    return _tc_compute(gp3, ipar, rat, up, npar, rating_table,
                       ln1_w, r1(ln1_b), ln2_w, r1(ln2_b), ln3_w, r1(ln3_b),
                       att1_w, r1(att1_b), att2_w, r1(att2_b), att3_w)


# speedup vs baseline: 2.2809x; 1.0146x over previous
import jax
import jax.numpy as jnp
from jax.experimental import pallas as pl
from jax.experimental.pallas import tpu as pltpu
from jax.experimental.pallas import tpu_sc as plsc


_NB = 128


def _sc_gather_items(table, idx, n_l, n_b, window):
    d = table.shape[1]
    n = idx.shape[0]
    per_l = n_b // window
    mesh = plsc.VectorSubcoreMesh(core_axis_name="core", subcore_axis_name="subcore")

    @pl.kernel(out_type=jax.ShapeDtypeStruct((n_l, n_b, d), table.dtype),
               mesh=mesh)
    def kern(tab_hbm, i_hbm, o_hbm):
        def body(i_vmem, o_vmem):
            pltpu.sync_copy(tab_hbm.at[i_vmem.at[0]], o_vmem.at[0])

        pltpu.emit_pipeline(
            body,
            grid=(n // window,),
            in_specs=[pl.BlockSpec((1, window), lambda i: (0, i))],
            out_specs=[pl.BlockSpec((1, window, d),
                                    lambda i: (i // per_l, i % per_l, 0))],
            core_axis_name=("core", "subcore"),
            dimension_semantics=(pltpu.PARALLEL,),
        )(i_hbm, o_hbm)

    return kern(table, idx.reshape(1, n))


def _sc_gather(table, idx, window):
    n = idx.shape[0]
    d = table.shape[1]
    mesh = plsc.VectorSubcoreMesh(core_axis_name="core", subcore_axis_name="subcore")

    @pl.kernel(out_type=jax.ShapeDtypeStruct((n, d), table.dtype), mesh=mesh)
    def kern(tab_hbm, i_hbm, o_hbm):
        def body(i_vmem, o_vmem):
            pltpu.sync_copy(tab_hbm.at[i_vmem.at[0]], o_vmem)

        pltpu.emit_pipeline(
            body,
            grid=(n // window,),
            in_specs=[pl.BlockSpec((1, window), lambda i: (0, i))],
            out_specs=[pl.BlockSpec((window, d), lambda i: (i, 0))],
            core_axis_name=("core", "subcore"),
            dimension_semantics=(pltpu.PARALLEL,),
        )(i_hbm, o_hbm)

    return kern(table, idx.reshape(1, n))


def _tc_body(gp_ref, ipar_ref, rat_ref, up_ref, npar_ref, rtab_ref,
             wi2_ref, wr_ref, ln1b_ref, wa_ref, wau2_ref, att1b_ref,
             att2w_ref, att2b_ref, att3w_ref, ln2w_ref, ln2b_ref,
             w3u2_ref, w3i_ref, ln3b_ref, out_ref):
    f32 = jnp.float32

    def dot_t(x, w):
        return jax.lax.dot_general(x, w, (((1,), (1,)), ((), ())),
                                   preferred_element_type=f32)

    L, nB, D2 = gp_ref.shape
    D = D2 // 2
    lane3 = jax.lax.broadcasted_iota(jnp.int32, (1, 1, D2), 2)
    hi3 = (lane3 >= D).astype(jnp.int32)
    gz = jnp.where((ipar_ref[...] & 1) == hi3, gp_ref[...], 0.0)
    item_part = dot_t(gz.reshape(L * nB, D2), wi2_ref[...])

    pre_rat = dot_t(rtab_ref[...], wr_ref[...]) + ln1b_ref[...]
    rat = rat_ref[...]
    ratc = jnp.zeros((L, nB, D), f32)
    for k in range(5):
        ratc = ratc + jnp.where(rat == k, pre_rat[k:k + 1][None], 0.0)
    xr3 = jnp.maximum(item_part.reshape(L, nB, D) + ratc, 0.0)
    xr2 = xr3.reshape(L * nB, D)

    lane2 = jax.lax.broadcasted_iota(jnp.int32, (1, D2), 1)
    hi2 = (lane2 >= D).astype(jnp.int32)
    uz = jnp.where((npar_ref[...] & 1) == hi2, up_ref[...], 0.0)
    u_att = dot_t(uz, wau2_ref[...]) + att1b_ref[...]
    a1 = dot_t(xr2, wa_ref[...]).reshape(L, nB, D) + u_att[None]
    a1 = jnp.maximum(a1, 0.0)
    a2 = jnp.maximum(dot_t(a1.reshape(L * nB, D), att2w_ref[...])
                     + att2b_ref[...], 0.0)

    s = jnp.sum(a2.reshape(L, nB, D) * att3w_ref[...][None], axis=2,
                keepdims=True)
    m = jnp.max(s, axis=0, keepdims=True)
    e = jnp.exp(s - m)
    denom = jnp.sum(e, axis=0, keepdims=True)
    hI = jnp.sum(xr3 * (e / denom), axis=0)

    h2 = jnp.maximum(dot_t(hI, ln2w_ref[...]) + ln2b_ref[...], 0.0)
    out = dot_t(uz, w3u2_ref[...]) + dot_t(h2, w3i_ref[...]) + ln3b_ref[...]
    out_ref[...] = jnp.maximum(out, 0.0)


def _tc_compute(gp3, ipar, rat, up, npar, rating_table, weights):
    L, B, D2 = gp3.shape
    grid = (B // _NB,)
    full = lambda a: pl.BlockSpec(a.shape, lambda i: tuple(0 for _ in a.shape))
    return pl.pallas_call(
        _tc_body,
        grid=grid,
        in_specs=[
            pl.BlockSpec((L, _NB, D2), lambda i: (0, i, 0)),
            pl.BlockSpec((L, _NB, 1), lambda i: (0, i, 0)),
            pl.BlockSpec((L, _NB, 1), lambda i: (0, i, 0)),
            pl.BlockSpec((_NB, D2), lambda i: (i, 0)),
            pl.BlockSpec((_NB, 1), lambda i: (i, 0)),
            full(rating_table),
        ] + [full(w) for w in weights],
        out_specs=pl.BlockSpec((_NB, D2 // 2), lambda i: (i, 0)),
        out_shape=jax.ShapeDtypeStruct((B, D2 // 2), jnp.float32),
        compiler_params=pltpu.CompilerParams(
            dimension_semantics=("parallel",)),
    )(gp3, ipar, rat, up, npar, rating_table, *weights)


def kernel(nodes, item_history, itemrating_history, user_table, item_table,
           rating_table, ln1_w, ln1_b, ln2_w, ln2_b, ln3_w, ln3_b,
           att1_w, att1_b, att2_w, att2_b, att3_w, att3_b):
    B, L = item_history.shape
    D = user_table.shape[1]

    idx_items = item_history.T.reshape(-1).astype(jnp.int32)
    idx_nodes = nodes.astype(jnp.int32)
    item_pairs = item_table.reshape(-1, 2 * D)
    user_pairs = user_table.reshape(-1, 2 * D)
    gp3 = _sc_gather_items(item_pairs, idx_items >> 1, L, B, 256)
    up = _sc_gather(user_pairs, idx_nodes >> 1, 128)

    ipar = idx_items.reshape(L, B, 1)
    rat = itemrating_history.T.astype(jnp.int32).reshape(L, B, 1)
    npar = idx_nodes.reshape(B, 1)

    r1 = lambda b: b.reshape(1, -1)
    double = lambda w: jnp.tile(w, (1, 2))
    weights = (
        double(ln1_w[:, :D]),
        ln1_w[:, D:],
        r1(ln1_b),
        att1_w[:, :D],
        double(att1_w[:, D:]),
        r1(att1_b),
        att2_w, r1(att2_b), att3_w,
        ln2_w, r1(ln2_b),
        double(ln3_w[:, :D]),
        ln3_w[:, D:],
        r1(ln3_b),
    )
    return _tc_compute(gp3, ipar, rat, up, npar, rating_table, weights)


# speedup vs baseline: 2.5730x; 1.1281x over previous
import jax
import jax.numpy as jnp
from jax.experimental import pallas as pl
from jax.experimental.pallas import tpu as pltpu
from jax.experimental.pallas import tpu_sc as plsc


_NB = 128


def _sc_gather_items(table, idx, n_l, n_b, window):
    d = table.shape[1]
    n = idx.shape[0]
    per_l = n_b // window
    mesh = plsc.VectorSubcoreMesh(core_axis_name="core", subcore_axis_name="subcore")

    @pl.kernel(out_type=jax.ShapeDtypeStruct((n_l, n_b, d), table.dtype),
               mesh=mesh)
    def kern(tab_hbm, i_hbm, o_hbm):
        def body(i_vmem, o_vmem):
            pltpu.sync_copy(tab_hbm.at[i_vmem.at[0]], o_vmem.at[0])

        pltpu.emit_pipeline(
            body,
            grid=(n // window,),
            in_specs=[pl.BlockSpec((1, window), lambda i: (0, i))],
            out_specs=[pl.BlockSpec((1, window, d),
                                    lambda i: (i // per_l, i % per_l, 0))],
            core_axis_name=("core", "subcore"),
            dimension_semantics=(pltpu.PARALLEL,),
        )(i_hbm, o_hbm)

    return kern(table, idx.reshape(1, n))


def _sc_gather_rows(table, idx):
    n = idx.shape[0]
    d = table.shape[1]
    mesh = plsc.ScalarSubcoreMesh(axis_name="core", num_cores=2)
    per = n // 2

    @pl.kernel(out_type=jax.ShapeDtypeStruct((n, d), table.dtype), mesh=mesh,
               scratch_types=[pltpu.SMEM((per,), jnp.int32),
                              pltpu.SemaphoreType.DMA,
                              pltpu.SemaphoreType.DMA])
    def kern(tab_hbm, i_hbm, o_hbm, idx_smem, sem_i, sem_d):
        c = jax.lax.axis_index("core")
        base = c * per
        pltpu.async_copy(i_hbm.at[0, pl.ds(base, per)], idx_smem, sem_i).wait()

        @pl.loop(0, per)
        def _(j):
            r = idx_smem[j]
            pltpu.async_copy(tab_hbm.at[pl.ds(r, 1), :],
                             o_hbm.at[pl.ds(base + j, 1), :], sem_d)

        @pl.loop(0, per)
        def _(j):
            pltpu.make_async_copy(tab_hbm.at[pl.ds(0, 1), :],
                                  o_hbm.at[pl.ds(0, 1), :], sem_d).wait()

    return kern(table, idx.reshape(1, n))


def _tc_body(gp_ref, ipar_ref, rat_ref, u_ref, rtab_ref,
             wi2_ref, wr_ref, ln1b_ref, wa_ref, wau_ref, att1b_ref,
             att2w_ref, att2b_ref, att3w_ref, ln2w_ref, ln2b_ref,
             w3u_ref, w3i_ref, ln3b_ref, out_ref):
    f32 = jnp.float32

    def dot_t(x, w):
        return jax.lax.dot_general(x, w, (((1,), (1,)), ((), ())),
                                   preferred_element_type=f32)

    L, nB, D2 = gp_ref.shape
    D = D2 // 2
    lane3 = jax.lax.broadcasted_iota(jnp.int32, (1, 1, D2), 2)
    hi3 = (lane3 >= D).astype(jnp.int32)
    gz = jnp.where((ipar_ref[...] & 1) == hi3, gp_ref[...], 0.0)
    item_part = dot_t(gz.reshape(L * nB, D2), wi2_ref[...])

    pre_rat = dot_t(rtab_ref[...], wr_ref[...]) + ln1b_ref[...]
    rat = rat_ref[...]
    ratc = jnp.zeros((L, nB, D), f32)
    for k in range(5):
        ratc = ratc + jnp.where(rat == k, pre_rat[k:k + 1][None], 0.0)
    xr3 = jnp.maximum(item_part.reshape(L, nB, D) + ratc, 0.0)
    xr2 = xr3.reshape(L * nB, D)

    u = u_ref[...]
    u_att = dot_t(u, wau_ref[...]) + att1b_ref[...]
    a1 = dot_t(xr2, wa_ref[...]).reshape(L, nB, D) + u_att[None]
    a1 = jnp.maximum(a1, 0.0)
    a2 = jnp.maximum(dot_t(a1.reshape(L * nB, D), att2w_ref[...])
                     + att2b_ref[...], 0.0)

    s = jnp.sum(a2.reshape(L, nB, D) * att3w_ref[...][None], axis=2,
                keepdims=True)
    m = jnp.max(s, axis=0, keepdims=True)
    e = jnp.exp(s - m)
    denom = jnp.sum(e, axis=0, keepdims=True)
    hI = jnp.sum(xr3 * (e / denom), axis=0)

    h2 = jnp.maximum(dot_t(hI, ln2w_ref[...]) + ln2b_ref[...], 0.0)
    out = dot_t(u, w3u_ref[...]) + dot_t(h2, w3i_ref[...]) + ln3b_ref[...]
    out_ref[...] = jnp.maximum(out, 0.0)


def _tc_compute(gp3, ipar, rat, u, rating_table, weights):
    L, B, D2 = gp3.shape
    grid = (B // _NB,)
    full = lambda a: pl.BlockSpec(a.shape, lambda i: tuple(0 for _ in a.shape))
    return pl.pallas_call(
        _tc_body,
        grid=grid,
        in_specs=[
            pl.BlockSpec((L, _NB, D2), lambda i: (0, i, 0)),
            pl.BlockSpec((L, _NB, 1), lambda i: (0, i, 0)),
            pl.BlockSpec((L, _NB, 1), lambda i: (0, i, 0)),
            pl.BlockSpec((_NB, D2 // 2), lambda i: (i, 0)),
            full(rating_table),
        ] + [full(w) for w in weights],
        out_specs=pl.BlockSpec((_NB, D2 // 2), lambda i: (i, 0)),
        out_shape=jax.ShapeDtypeStruct((B, D2 // 2), jnp.float32),
        compiler_params=pltpu.CompilerParams(
            dimension_semantics=("parallel",)),
    )(gp3, ipar, rat, u, rating_table, *weights)


def kernel(nodes, item_history, itemrating_history, user_table, item_table,
           rating_table, ln1_w, ln1_b, ln2_w, ln2_b, ln3_w, ln3_b,
           att1_w, att1_b, att2_w, att2_b, att3_w, att3_b):
    B, L = item_history.shape
    D = user_table.shape[1]

    idx_items = item_history.T.reshape(-1).astype(jnp.int32)
    idx_nodes = nodes.astype(jnp.int32)
    item_pairs = item_table.reshape(-1, 2 * D)
    gp3 = _sc_gather_items(item_pairs, idx_items >> 1, L, B, 256)
    u = _sc_gather_rows(user_table, idx_nodes)

    ipar = idx_items.reshape(L, B, 1)
    rat = itemrating_history.T.astype(jnp.int32).reshape(L, B, 1)

    r1 = lambda b: b.reshape(1, -1)
    double = lambda w: jnp.tile(w, (1, 2))
    weights = (
        double(ln1_w[:, :D]),
        ln1_w[:, D:],
        r1(ln1_b),
        att1_w[:, :D],
        att1_w[:, D:],
        r1(att1_b),
        att2_w, r1(att2_b), att3_w,
        ln2_w, r1(ln2_b),
        ln3_w[:, :D],
        ln3_w[:, D:],
        r1(ln3_b),
    )
    return _tc_compute(gp3, ipar, rat, u, rating_table, weights)


# speedup vs baseline: 2.6070x; 1.0132x over previous
import jax
import jax.numpy as jnp
from jax.experimental import pallas as pl
from jax.experimental.pallas import tpu as pltpu
from jax.experimental.pallas import tpu_sc as plsc


_NB = 128


def _sc_gather_items(table, idx, n_l, n_b, window):
    d = table.shape[1]
    n = idx.shape[0]
    per_l = n_b // window
    mesh = plsc.VectorSubcoreMesh(core_axis_name="core", subcore_axis_name="subcore")

    @pl.kernel(out_type=jax.ShapeDtypeStruct((n_l, n_b, d), table.dtype),
               mesh=mesh)
    def kern(tab_hbm, i_hbm, o_hbm):
        def body(i_vmem, o_vmem):
            pltpu.sync_copy(tab_hbm.at[i_vmem.at[0]], o_vmem.at[0])

        pltpu.emit_pipeline(
            body,
            grid=(n // window,),
            in_specs=[pl.BlockSpec((1, window), lambda i: (0, i))],
            out_specs=[pl.BlockSpec((1, window, d),
                                    lambda i: (i // per_l, i % per_l, 0))],
            core_axis_name=("core", "subcore"),
            dimension_semantics=(pltpu.PARALLEL,),
        )(i_hbm, o_hbm)

    return kern(table, idx.reshape(1, n))


def _tc_body(gp_ref, rat_ref, nodes_ref, utab_ref, rtab_ref,
             wi2_ref, wr_ref, ln1b_ref, wa_ref, wau_ref, att1b_ref,
             att2w_ref, att2b_ref, att3w_ref, ln2w_ref, ln2b_ref,
             w3u_ref, w3i_ref, ln3b_ref, out_ref, u_scr, u_sem):
    f32 = jnp.float32

    def dot_t(x, w):
        return jax.lax.dot_general(x, w, (((1,), (1,)), ((), ())),
                                   preferred_element_type=f32)

    L, nB, D2 = gp_ref.shape
    D = D2 // 2

    def issue(j, _):
        r = nodes_ref[0, 0, j]
        pltpu.make_async_copy(utab_ref.at[pl.ds(r, 1), :],
                              u_scr.at[pl.ds(j, 1), :], u_sem).start()
        return _
    jax.lax.fori_loop(0, nB, issue, None)

    item_part = dot_t(gp_ref[...].reshape(L * nB, D2), wi2_ref[...])

    pre_rat = dot_t(rtab_ref[...], wr_ref[...]) + ln1b_ref[...]
    p = [pre_rat[k:k + 1][None] for k in range(5)]
    r3 = rat_ref[...]
    b0 = (r3 & 1) == 1
    v01 = jnp.where(b0, p[1], p[0])
    v23 = jnp.where(b0, p[3], p[2])
    v0123 = jnp.where((r3 & 2) == 2, v23, v01)
    ratc = jnp.where(r3 >= 4, p[4], v0123)
    xr3 = jnp.maximum(item_part.reshape(L, nB, D) + ratc, 0.0)
    xr2 = xr3.reshape(L * nB, D)

    def drain(j, _):
        pltpu.make_async_copy(utab_ref.at[pl.ds(0, 1), :],
                              u_scr.at[pl.ds(0, 1), :], u_sem).wait()
        return _
    jax.lax.fori_loop(0, nB, drain, None)
    u = u_scr[...]

    u_att = dot_t(u, wau_ref[...]) + att1b_ref[...]
    a1 = dot_t(xr2, wa_ref[...]).reshape(L, nB, D) + u_att[None]
    a1 = jnp.maximum(a1, 0.0)
    a2 = jnp.maximum(dot_t(a1.reshape(L * nB, D), att2w_ref[...])
                     + att2b_ref[...], 0.0)

    s = jnp.sum(a2.reshape(L, nB, D) * att3w_ref[...][None], axis=2,
                keepdims=True)
    m = jnp.max(s, axis=0, keepdims=True)
    e = jnp.exp(s - m)
    denom = jnp.sum(e, axis=0, keepdims=True)
    hI = jnp.sum(xr3 * (e / denom), axis=0)

    h2 = jnp.maximum(dot_t(hI, ln2w_ref[...]) + ln2b_ref[...], 0.0)
    out = dot_t(u, w3u_ref[...]) + dot_t(h2, w3i_ref[...]) + ln3b_ref[...]
    out_ref[...] = jnp.maximum(out, 0.0)


def _tc_compute(gp3, rat, nodes_b, user_table, rating_table, weights):
    L, B, D2 = gp3.shape
    D = D2 // 2
    grid = (B // _NB,)
    full = lambda a: pl.BlockSpec(a.shape, lambda i: tuple(0 for _ in a.shape))
    return pl.pallas_call(
        _tc_body,
        grid=grid,
        in_specs=[
            pl.BlockSpec((L, _NB, D2), lambda i: (0, i, 0)),
            pl.BlockSpec((L, _NB, 1), lambda i: (0, i, 0)),
            pl.BlockSpec((1, 1, _NB), lambda i: (i, 0, 0),
                         memory_space=pltpu.SMEM),
            pl.BlockSpec(memory_space=pl.ANY),
            full(rating_table),
        ] + [full(w) for w in weights],
        out_specs=pl.BlockSpec((_NB, D), lambda i: (i, 0)),
        out_shape=jax.ShapeDtypeStruct((B, D), jnp.float32),
        scratch_shapes=[pltpu.VMEM((_NB, D), jnp.float32),
                        pltpu.SemaphoreType.DMA],
        compiler_params=pltpu.CompilerParams(
            dimension_semantics=("parallel",)),
    )(gp3, rat, nodes_b, user_table, rating_table, *weights)


def kernel(nodes, item_history, itemrating_history, user_table, item_table,
           rating_table, ln1_w, ln1_b, ln2_w, ln2_b, ln3_w, ln3_b,
           att1_w, att1_b, att2_w, att2_b, att3_w, att3_b):
    B, L = item_history.shape
    D = user_table.shape[1]

    idx_items = item_history.T.reshape(-1).astype(jnp.int32)
    item_dup = jnp.concatenate([item_table, item_table], axis=1)
    gp3 = _sc_gather_items(item_dup, idx_items, L, B, 256)

    rat = itemrating_history.T.astype(jnp.int32).reshape(L, B, 1)
    nodes_b = nodes.astype(jnp.int32).reshape(B // _NB, 1, _NB)

    r1 = lambda b: b.reshape(1, -1)
    weights = (
        jnp.tile(0.5 * ln1_w[:, :D], (1, 2)),
        ln1_w[:, D:],
        r1(ln1_b),
        att1_w[:, :D],
        att1_w[:, D:],
        r1(att1_b),
        att2_w, r1(att2_b), att3_w,
        ln2_w, r1(ln2_b),
        ln3_w[:, :D],
        ln3_w[:, D:],
        r1(ln3_b),
    )
    return _tc_compute(gp3, rat, nodes_b, user_table, rating_table, weights)
